# baseline (speedup 1.0000x reference)
import functools

import jax
import jax.numpy as jnp
from jax import lax
from jax.experimental import pallas as pl
from jax.experimental.pallas import tpu as pltpu
from jax.experimental.pallas import tpu_sc as plsc

B = 32
P = 20000
C = 5
ROWP = 20224
NC = 2
NS = 16
L = 16
CHS = 4000
NCH = P // CHS
GRP = CHS // L
RPB = 8


def _prep_body(ct_ref, lt_ref, tlt_ref, lab_ref, m_ref, wpart_ref, lpart_ref):
    lab = lab_ref[...]
    pos = lab > 0
    posf = jnp.where(pos, 1.0, 0.0)
    x = [ct_ref[c] for c in range(C)]
    mx = jnp.maximum(jnp.maximum(jnp.maximum(x[0], x[1]),
                                 jnp.maximum(x[2], x[3])), x[4])
    s = (jnp.exp(x[0] - mx) + jnp.exp(x[1] - mx) + jnp.exp(x[2] - mx)
         + jnp.exp(x[3] - mx) + jnp.exp(x[4] - mx))
    lse = mx + jnp.log(s)
    xl = jnp.where(lab == 1, x[1], x[0])
    xl = jnp.where(lab == 2, x[2], xl)
    xl = jnp.where(lab == 3, x[3], xl)
    xl = jnp.where(lab == 4, x[4], xl)
    w = jnp.where(lab == 1, 1.0, 0.05)
    w = jnp.where(lab == 2, 5.0, w)
    w = jnp.where(lab == 3, 20.0, w)
    w = jnp.where(lab == 4, 10.0, w)
    wtot = jnp.sum(posf * (w * (lse - xl)))
    mvals = jnp.where(pos, -1.0, lse - x[0])
    mp = jnp.concatenate(
        [mvals, jnp.full((RPB, ROWP - P), -1.0, jnp.float32)],
        axis=1).astype(jnp.bfloat16)
    for r in range(RPB):
        m_ref[pl.ds(r * ROWP, ROWP)] = mp[r, :]
    d = lt_ref[...] - tlt_ref[...]
    sq = jnp.sum(d * d, axis=1)
    e = jnp.abs(jnp.exp(lt_ref[:, 2:4, :]) - jnp.exp(tlt_ref[:, 2:4, :]))
    ltot = jnp.sum((sq + jnp.sum(e, axis=1)) * posf)
    lane0 = jax.lax.broadcasted_iota(jnp.int32, (1, 1, 128), 2) == 0
    wpart_ref[...] = jnp.where(lane0, wtot, 0.0)
    lpart_ref[...] = jnp.where(lane0, ltot, 0.0)


def _prep(ct, lt, tlt, lab):
    nsteps = B // RPB
    return pl.pallas_call(
        _prep_body,
        grid=(nsteps,),
        in_specs=[
            pl.BlockSpec((C, RPB, P), lambda i: (0, i, 0)),
            pl.BlockSpec((RPB, 4, P), lambda i: (i, 0, 0)),
            pl.BlockSpec((RPB, 4, P), lambda i: (i, 0, 0)),
            pl.BlockSpec((RPB, P), lambda i: (i, 0)),
        ],
        out_specs=[
            pl.BlockSpec((RPB * ROWP,), lambda i: (i,)),
            pl.BlockSpec((1, 1, 128), lambda i: (i, 0, 0)),
            pl.BlockSpec((1, 1, 128), lambda i: (i, 0, 0)),
        ],
        out_shape=[
            jax.ShapeDtypeStruct((B * ROWP,), jnp.bfloat16),
            jax.ShapeDtypeStruct((nsteps, 1, 128), jnp.float32),
            jax.ShapeDtypeStruct((nsteps, 1, 128), jnp.float32),
        ],
    )(ct, lt, tlt, lab)


_UNROLL = 2
_NITER = ROWP // (2 * L * _UNROLL)


def _sc_body(m_hbm, part_hbm, mbuf, obuf):
    wid = lax.axis_index("s") * NC + lax.axis_index("c")
    zero = jnp.zeros((L,), jnp.float32)
    pltpu.sync_copy(m_hbm.at[pl.ds(wid * ROWP, ROWP)], mbuf)

    def group(g, carry):
        base = g * (2 * L * _UNROLL)
        out = []
        for u in range(_UNROLL):
            npos_u, negsum_u = carry[2 * u], carry[2 * u + 1]
            vb = mbuf[pl.ds(base + u * (2 * L), 2 * L)]
            va, vc = plsc.unpack(vb, format=plsc.PackFormat.INTERLEAVED)
            for v in (va, vc):
                isneg = v >= 0.0
                npos_u = npos_u + jnp.where(isneg, 0.0, 1.0)
                negsum_u = negsum_u + jnp.where(isneg, v, 0.0)
            out.append(npos_u)
            out.append(negsum_u)
        return tuple(out)

    acc = lax.fori_loop(0, _NITER, group, (zero,) * (2 * _UNROLL))
    npos_a = acc[0] + acc[2]
    negsum_a = acc[1] + acc[3]
    obuf[pl.ds(0, L)] = npos_a
    obuf[pl.ds(L, L)] = negsum_a
    pltpu.sync_copy(obuf, part_hbm.at[pl.ds(wid * (2 * L), 2 * L)])


@functools.cache
def _sc_main():
    return pl.kernel(
        _sc_body,
        out_type=jax.ShapeDtypeStruct((B * 2 * L,), jnp.float32),
        mesh=plsc.VectorSubcoreMesh(core_axis_name="c", subcore_axis_name="s",
                                    num_cores=NC, num_subcores=NS),
        scratch_types=[
            pltpu.VMEM((ROWP,), jnp.bfloat16),
            pltpu.VMEM((2 * L,), jnp.float32),
        ],
        compiler_params=pltpu.CompilerParams(needs_layout_passes=False),
    )


def _fin_body(part_ref, wpart_ref, lpart_ref, out_ref, npr_ref):
    part = part_ref[...]
    npos_rows = jnp.sum(part[:, :L], axis=1, keepdims=True) - float(ROWP - P)
    num_pos = jnp.sum(npos_rows)
    negsum = jnp.sum(part[:, L:])
    base = jnp.sum(wpart_ref[...]) + jnp.sum(lpart_ref[...])
    common = (base + 0.05 * negsum) / num_pos
    i = jax.lax.broadcasted_iota(jnp.int32, (1, 128), 1)
    out_ref[...] = (jnp.where(i == 0, common, 0.0)
                    + jnp.where(i == 1, base, 0.0)
                    + jnp.where(i == 2, num_pos, 0.0))
    npr_ref[...] = npos_rows


def _finalize(partials, wpart, lpart):
    return pl.pallas_call(
        _fin_body,
        out_shape=[
            jax.ShapeDtypeStruct((1, 128), jnp.float32),
            jax.ShapeDtypeStruct((B, 1), jnp.float32),
        ],
    )(partials.reshape(B, 2 * L), wpart, lpart)


def _topk_body(m_ref, k_ref, nneg_ref, out_ref):
    v = m_ref[...].astype(jnp.float32)
    neg = v >= 0.0
    k_eff = jnp.minimum(k_ref[0, 0], nneg_ref[...])

    def bit_step(i, t_bits):
        cand = jnp.bitwise_or(t_bits, lax.shift_left(jnp.int32(1), 30 - i))
        t = lax.bitcast_convert_type(cand, jnp.float32)
        cnt = jnp.sum(jnp.where(neg & (v >= t), 1.0, 0.0), axis=1, keepdims=True)
        return jnp.where(cnt >= k_eff, cand, t_bits)

    t_bits = lax.fori_loop(0, 31, bit_step, jnp.zeros((B, 1), jnp.int32))
    t = lax.bitcast_convert_type(t_bits, jnp.float32)
    sel = neg & (v > t)
    cgt = jnp.sum(jnp.where(sel, 1.0, 0.0), axis=1, keepdims=True)
    ssum = jnp.sum(jnp.where(sel, v, 0.0), axis=1, keepdims=True)
    rows = ssum + (k_eff - cgt) * t
    rows = jnp.where(k_eff > 0.0, rows, 0.0)
    out_ref[...] = jnp.sum(rows).reshape(1, 1)


def _topk_sum(m2d, kf, nneg):
    return pl.pallas_call(
        _topk_body,
        out_shape=jax.ShapeDtypeStruct((1, 1), jnp.float32),
    )(m2d, kf.reshape(1, 1), nneg.reshape(B, 1))[0, 0]


def kernel(confidence, locations, target_confidence, target_locations):
    ct = jnp.transpose(confidence, (2, 0, 1))
    lt = jnp.transpose(locations, (0, 2, 1))
    tlt = jnp.transpose(target_locations, (0, 2, 1))
    m1d, wpart, lpart = _prep(ct, lt, tlt, target_confidence)
    partials = _sc_main()(m1d)
    scals, npr = _finalize(partials, wpart, lpart)
    common, base, num_pos = scals[0, 0], scals[0, 1], scals[0, 2]
    kf = 3.0 * num_pos
    return lax.cond(
        kf >= float(P),
        lambda: common,
        lambda: (base + 0.05 * _topk_sum(m1d.reshape(B, ROWP), kf,
                                         float(P) - npr[:, 0])) / num_pos,
    )


# speedup vs baseline: 1.0462x; 1.0462x over previous
import functools

import jax
import jax.numpy as jnp
from jax import lax
from jax.experimental import pallas as pl
from jax.experimental.pallas import tpu as pltpu
from jax.experimental.pallas import tpu_sc as plsc

B = 32
P = 20000
C = 5
ROWP = 20224
NC = 2
NS = 16
L = 16
CHS = 4000
NCH = P // CHS
GRP = CHS // L
RPB = 8


def _prep_body(ct_ref, lt_ref, tlt_ref, lab_ref, m_ref, wpart_ref, lpart_ref):
    lab = lab_ref[...]
    pos = lab > 0
    posf = jnp.where(pos, 1.0, 0.0)
    x = [ct_ref[c] for c in range(C)]
    mx = jnp.maximum(jnp.maximum(jnp.maximum(x[0], x[1]),
                                 jnp.maximum(x[2], x[3])), x[4])
    s = (jnp.exp(x[0] - mx) + jnp.exp(x[1] - mx) + jnp.exp(x[2] - mx)
         + jnp.exp(x[3] - mx) + jnp.exp(x[4] - mx))
    lse = mx + jnp.log(s)
    xl = jnp.where(lab == 1, x[1], x[0])
    xl = jnp.where(lab == 2, x[2], xl)
    xl = jnp.where(lab == 3, x[3], xl)
    xl = jnp.where(lab == 4, x[4], xl)
    w = jnp.where(lab == 1, 1.0, 0.05)
    w = jnp.where(lab == 2, 5.0, w)
    w = jnp.where(lab == 3, 20.0, w)
    w = jnp.where(lab == 4, 10.0, w)
    wtot = jnp.sum(posf * (w * (lse - xl)))
    mvals = jnp.where(pos, -1.0, lse - x[0])
    mp = jnp.concatenate(
        [mvals, jnp.full((RPB, ROWP - P), -1.0, jnp.float32)],
        axis=1).astype(jnp.bfloat16)
    for r in range(RPB):
        m_ref[pl.ds(r * ROWP, ROWP)] = mp[r, :]
    d = lt_ref[...] - tlt_ref[...]
    sq = jnp.sum(d * d, axis=1)
    e = jnp.abs(jnp.exp(lt_ref[:, 2:4, :]) - jnp.exp(tlt_ref[:, 2:4, :]))
    ltot = jnp.sum((sq + jnp.sum(e, axis=1)) * posf)
    lane0 = jax.lax.broadcasted_iota(jnp.int32, (1, 1, 128), 2) == 0
    wpart_ref[...] = jnp.where(lane0, wtot, 0.0)
    lpart_ref[...] = jnp.where(lane0, ltot, 0.0)


def _prep(ct, lt, tlt, lab):
    nsteps = B // RPB
    return pl.pallas_call(
        _prep_body,
        grid=(nsteps,),
        in_specs=[
            pl.BlockSpec((C, RPB, P), lambda i: (0, i, 0)),
            pl.BlockSpec((RPB, 4, P), lambda i: (i, 0, 0)),
            pl.BlockSpec((RPB, 4, P), lambda i: (i, 0, 0)),
            pl.BlockSpec((RPB, P), lambda i: (i, 0)),
        ],
        out_specs=[
            pl.BlockSpec((RPB * ROWP,), lambda i: (i,)),
            pl.BlockSpec((1, 1, 128), lambda i: (i, 0, 0)),
            pl.BlockSpec((1, 1, 128), lambda i: (i, 0, 0)),
        ],
        out_shape=[
            jax.ShapeDtypeStruct((B * ROWP,), jnp.bfloat16),
            jax.ShapeDtypeStruct((nsteps, 1, 128), jnp.float32),
            jax.ShapeDtypeStruct((nsteps, 1, 128), jnp.float32),
        ],
    )(ct, lt, tlt, lab)


_UNROLL = 2
_NITER = ROWP // (2 * L * _UNROLL)


def _sc_body(m_hbm, part_hbm, mbuf, obuf):
    wid = lax.axis_index("s") * NC + lax.axis_index("c")
    zero = jnp.zeros((L,), jnp.float32)
    pltpu.sync_copy(m_hbm.at[pl.ds(wid * ROWP, ROWP)], mbuf)

    def group(g, carry):
        base = g * (2 * L * _UNROLL)
        out = []
        for u in range(_UNROLL):
            npos_u, negsum_u = carry[2 * u], carry[2 * u + 1]
            vb = mbuf[pl.ds(base + u * (2 * L), 2 * L)]
            va, vc = plsc.unpack(vb, format=plsc.PackFormat.INTERLEAVED)
            for v in (va, vc):
                isneg = v >= 0.0
                npos_u = npos_u + jnp.where(isneg, 0.0, 1.0)
                negsum_u = negsum_u + jnp.where(isneg, v, 0.0)
            out.append(npos_u)
            out.append(negsum_u)
        return tuple(out)

    acc = lax.fori_loop(0, _NITER, group, (zero,) * (2 * _UNROLL))
    npos_a = acc[0] + acc[2]
    negsum_a = acc[1] + acc[3]
    obuf[pl.ds(0, L)] = npos_a
    obuf[pl.ds(L, L)] = negsum_a
    pltpu.sync_copy(obuf, part_hbm.at[pl.ds(wid * (2 * L), 2 * L)])


@functools.cache
def _sc_main():
    return pl.kernel(
        _sc_body,
        out_type=jax.ShapeDtypeStruct((B * 2 * L,), jnp.float32),
        mesh=plsc.VectorSubcoreMesh(core_axis_name="c", subcore_axis_name="s",
                                    num_cores=NC, num_subcores=NS),
        scratch_types=[
            pltpu.VMEM((ROWP,), jnp.bfloat16),
            pltpu.VMEM((2 * L,), jnp.float32),
        ],
        compiler_params=pltpu.CompilerParams(needs_layout_passes=False),
    )


def _topk_body(m_ref, k_ref, nneg_ref, out_ref):
    v = m_ref[...].astype(jnp.float32)
    neg = v >= 0.0
    k_eff = jnp.minimum(k_ref[0, 0], nneg_ref[...])

    def bit_step(i, t_bits):
        cand = jnp.bitwise_or(t_bits, lax.shift_left(jnp.int32(1), 30 - i))
        t = lax.bitcast_convert_type(cand, jnp.float32)
        cnt = jnp.sum(jnp.where(neg & (v >= t), 1.0, 0.0), axis=1, keepdims=True)
        return jnp.where(cnt >= k_eff, cand, t_bits)

    t_bits = lax.fori_loop(0, 31, bit_step, jnp.zeros((B, 1), jnp.int32))
    t = lax.bitcast_convert_type(t_bits, jnp.float32)
    sel = neg & (v > t)
    cgt = jnp.sum(jnp.where(sel, 1.0, 0.0), axis=1, keepdims=True)
    ssum = jnp.sum(jnp.where(sel, v, 0.0), axis=1, keepdims=True)
    rows = ssum + (k_eff - cgt) * t
    rows = jnp.where(k_eff > 0.0, rows, 0.0)
    out_ref[...] = jnp.sum(rows).reshape(1, 1)


def _topk_sum(m2d, kf, nneg):
    return pl.pallas_call(
        _topk_body,
        out_shape=jax.ShapeDtypeStruct((1, 1), jnp.float32),
    )(m2d, kf.reshape(1, 1), nneg.reshape(B, 1))[0, 0]


def kernel(confidence, locations, target_confidence, target_locations):
    ct = jnp.transpose(confidence, (2, 0, 1))
    lt = jnp.transpose(locations, (0, 2, 1))
    tlt = jnp.transpose(target_locations, (0, 2, 1))
    m1d, wpart, lpart = _prep(ct, lt, tlt, target_confidence)
    partials = _sc_main()(m1d)
    part = partials.reshape(B, 2, L)
    npos_rows = jnp.sum(part[:, 0, :], axis=1) - float(ROWP - P)
    num_pos = jnp.sum(npos_rows)
    negsum_rows = jnp.sum(part[:, 1, :], axis=1)
    wsum = jnp.sum(wpart)
    locsum = jnp.sum(lpart)
    kf = 3.0 * num_pos
    neg_contrib = lax.cond(
        kf >= float(P),
        lambda: jnp.sum(negsum_rows),
        lambda: _topk_sum(m1d.reshape(B, ROWP), kf, float(P) - npos_rows),
    )
    return (locsum + wsum + 0.05 * neg_contrib) / num_pos
